# baseline (speedup 1.0000x reference)
import jax
import jax.numpy as jnp
from jax.experimental import pallas as pl

_J = 100
_M = 20
_N = _J + _M
_H = 3
_F0 = 16
_ED = 128


def _lrelu(x, s):
    return jnp.maximum(x, s * x)


def _gat(ftall, al, ar, ae, we, b, F, penalty, scale_guard, Tp, ones_src):
    ft_src = ftall[:_J, :]
    acc = jnp.zeros((_N, F), jnp.float32)
    for h in range(_H):
        fth = ftall[:, h * F:(h + 1) * F]
        fsh = ft_src[:, h * F:(h + 1) * F]
        alh = al[h:h + 1, :]
        arh = ar[h:h + 1, :]
        weh = we[h:h + 1, :]
        el = jax.lax.dot_general(fsh, alh, (((1,), (1,)), ((), ())),
                                 preferred_element_type=jnp.float32)
        er = jax.lax.dot_general(arh, fth, (((1,), (1,)), ((), ())),
                                 preferred_element_type=jnp.float32)
        eec = jnp.sum(weh * ae[h:h + 1, :])
        lg = _lrelu(el + er + Tp * eec, 0.2) + penalty
        mx = jnp.max(lg, axis=0, keepdims=True)
        ex = jnp.exp(lg - mx)
        den = jax.lax.dot_general(ex, ones_src, (((0,), (0,)), ((), ())),
                                  preferred_element_type=jnp.float32)
        num = jax.lax.dot_general(ex, fsh, (((0,), (0,)), ((), ())),
                                  preferred_element_type=jnp.float32)
        eagg = jax.lax.dot_general(ex * Tp, ones_src,
                                   (((0,), (0,)), ((), ())),
                                   preferred_element_type=jnp.float32)
        scale = scale_guard / den
        outh = (num + eagg * weh) * scale + b[h:h + 1, :]
        acc = acc + _lrelu(outh, 0.01)
    return acc * (1.0 / _H)


def _gnn_body(feat_ref, g_ref, t_ref, lng_ref, lnb_ref, w0_ref, al0_ref,
              ar0_ref, ae0_ref, we0_ref, b0_ref, w1_ref, al1_ref, ar1_ref,
              ae1_ref, we1_ref, b1_ref, out_ref):
    f = feat_ref[0]
    lane = jax.lax.broadcasted_iota(jnp.int32, (_N, 8), 1)
    feat_on = lane < 5
    fm = jnp.where(feat_on, f, 0.0)
    mu = jnp.sum(fm, axis=1, keepdims=True) * 0.2
    var = jnp.sum(jnp.where(feat_on, (fm - mu) ** 2, 0.0),
                  axis=1, keepdims=True) * 0.2
    nf = (fm - mu) * jax.lax.rsqrt(var + 1e-5) * lng_ref[...] + lnb_ref[...]
    nf = jnp.where(feat_on, nf, 0.0)

    g = g_ref[0]
    penalty = (g - 1.0) * 1e9
    Tp = jnp.concatenate(
        [t_ref[0], jnp.zeros((_J, _N - _J), jnp.float32)], axis=1)
    ones_src = jnp.ones((_J, 1), jnp.float32)
    deg = jax.lax.dot_general(g, ones_src, (((0,), (0,)), ((), ())),
                              preferred_element_type=jnp.float32)
    scale_guard = jnp.where(deg > 0, 1.0, 0.0)

    ft0 = jnp.dot(nf, w0_ref[...], preferred_element_type=jnp.float32)
    h0 = _gat(ft0, al0_ref[...], ar0_ref[...], ae0_ref[...], we0_ref[...],
              b0_ref[...], _F0, penalty, scale_guard, Tp, ones_src)
    ft1 = jnp.dot(h0, w1_ref[...], preferred_element_type=jnp.float32)
    h1 = _gat(ft1, al1_ref[...], ar1_ref[...], ae1_ref[...], we1_ref[...],
              b1_ref[...], _ED, penalty, scale_guard, Tp, ones_src)
    out_ref[0] = h1


def _final_body(x_ref, w_ref, b_ref, out_ref):
    k = pl.program_id(0)
    acc = jnp.zeros((x_ref.shape[0], _ED), jnp.float32)
    for n in range(8):
        acc = acc + jnp.dot(x_ref[:, n, :], w_ref[n],
                            preferred_element_type=jnp.float32)

    @pl.when(k == 0)
    def _():
        out_ref[...] = acc

    @pl.when(k > 0)
    def _():
        out_ref[...] = out_ref[...] + acc

    @pl.when(k == (_N // 8) - 1)
    def _():
        out_ref[...] = _lrelu(out_ref[...] + b_ref[...], 0.01)


def kernel(Graph, norm_h, norm_L, norm_W, norm_P, norm_N, T, ln_g, ln_b,
           W0, We0, al0, ar0, ae0, b0, W1, We1, al1, ar1, ae1, b1, Wl, bl):
    bs = Graph.shape[0]
    G = Graph.reshape(bs, _J, _N)
    other = jnp.concatenate([norm_W, norm_P, norm_N], axis=1)
    jobf = jnp.concatenate(
        [norm_h[:, :, None], norm_L[:, :, None],
         jnp.broadcast_to(other[:, None, :], (bs, _J, 3))], axis=2)
    feats = jnp.pad(jobf, ((0, 0), (0, _N - _J), (0, 3)))
    lng = jnp.pad(ln_g, (0, 3)).reshape(1, 8)
    lnb = jnp.pad(ln_b, (0, 3)).reshape(1, 8)
    w0 = jnp.pad(W0, ((0, 3), (0, 0)))
    we0 = We0.reshape(_H, _F0)
    b0r = b0.reshape(_H, _F0)
    we1 = We1.reshape(_H, _ED)
    b1r = b1.reshape(_H, _ED)

    h1 = pl.pallas_call(
        _gnn_body,
        grid=(bs,),
        in_specs=[
            pl.BlockSpec((1, _N, 8), lambda i: (i, 0, 0)),
            pl.BlockSpec((1, _J, _N), lambda i: (i, 0, 0)),
            pl.BlockSpec((1, _J, _J), lambda i: (i, 0, 0)),
            pl.BlockSpec((1, 8), lambda i: (0, 0)),
            pl.BlockSpec((1, 8), lambda i: (0, 0)),
            pl.BlockSpec((8, _H * _F0), lambda i: (0, 0)),
            pl.BlockSpec((_H, _F0), lambda i: (0, 0)),
            pl.BlockSpec((_H, _F0), lambda i: (0, 0)),
            pl.BlockSpec((_H, _F0), lambda i: (0, 0)),
            pl.BlockSpec((_H, _F0), lambda i: (0, 0)),
            pl.BlockSpec((_H, _F0), lambda i: (0, 0)),
            pl.BlockSpec((_F0, _H * _ED), lambda i: (0, 0)),
            pl.BlockSpec((_H, _ED), lambda i: (0, 0)),
            pl.BlockSpec((_H, _ED), lambda i: (0, 0)),
            pl.BlockSpec((_H, _ED), lambda i: (0, 0)),
            pl.BlockSpec((_H, _ED), lambda i: (0, 0)),
            pl.BlockSpec((_H, _ED), lambda i: (0, 0)),
        ],
        out_specs=pl.BlockSpec((1, _N, _ED), lambda i: (i, 0, 0)),
        out_shape=jax.ShapeDtypeStruct((bs, _N, _ED), jnp.float32),
    )(feats, G, T, lng, lnb, w0, al0, ar0, ae0, we0, b0r,
      W1, al1, ar1, ae1, we1, b1r)

    Wlr = Wl.reshape(_N, _ED, _ED)
    out = pl.pallas_call(
        _final_body,
        grid=(_N // 8,),
        in_specs=[
            pl.BlockSpec((bs, 8, _ED), lambda k: (0, k, 0)),
            pl.BlockSpec((8, _ED, _ED), lambda k: (k, 0, 0)),
            pl.BlockSpec((1, _ED), lambda k: (0, 0)),
        ],
        out_specs=pl.BlockSpec((bs, _ED), lambda k: (0, 0)),
        out_shape=jax.ShapeDtypeStruct((bs, _ED), jnp.float32),
    )(h1, Wlr, bl.reshape(1, _ED))
    return out


# speedup vs baseline: 1.3852x; 1.3852x over previous
import jax
import jax.numpy as jnp
from jax.experimental import pallas as pl

_J = 100
_M = 20
_N = _J + _M
_H = 3
_F0 = 16
_ED = 128
_SP = 128
_B = 2


def _lrelu(x, s):
    return jnp.where(x >= 0, x, s * x)


def _gat(ftall, al, ar, ae, we, b, F, mask, Tm, ones_col):
    acc = jnp.zeros((_SP, F), jnp.float32)
    for h in range(_H):
        fth = ftall[:, h * F:(h + 1) * F]
        alh = al[h:h + 1, :]
        arh = ar[h:h + 1, :]
        weh = we[h:h + 1, :]
        el = jax.lax.dot_general(fth, alh, (((1,), (1,)), ((), ())),
                                 preferred_element_type=jnp.float32)
        er = jax.lax.dot_general(arh, fth, (((1,), (1,)), ((), ())),
                                 preferred_element_type=jnp.float32)
        eec = jnp.sum(weh * ae[h:h + 1, :])
        lg = _lrelu(el + er + Tm * eec, 0.2)
        lg = jnp.where(mask, lg, -1e9)
        mx = jnp.max(lg, axis=0, keepdims=True)
        ex = jnp.where(mask, jnp.exp(lg - mx), 0.0)
        den = jnp.sum(ex, axis=0, keepdims=True)
        alpha = ex / jnp.where(den > 0, den, 1.0)
        outh = jax.lax.dot_general(alpha, fth, (((0,), (0,)), ((), ())),
                                   preferred_element_type=jnp.float32)
        eagg = jax.lax.dot_general(alpha * Tm, ones_col,
                                   (((0,), (0,)), ((), ())),
                                   preferred_element_type=jnp.float32)
        outh = outh + eagg * weh + b[h:h + 1, :]
        acc = acc + _lrelu(outh, 0.01)
    return acc * (1.0 / _H)


def _gnn_body(feat_ref, g_ref, t_ref, lng_ref, lnb_ref, w0_ref, al0_ref,
              ar0_ref, ae0_ref, we0_ref, b0_ref, w1_ref, al1_ref, ar1_ref,
              ae1_ref, we1_ref, b1_ref, out_ref):
    lane = jax.lax.broadcasted_iota(jnp.int32, (_SP, 8), 1)
    feat_on = lane < 5
    ones_col = jnp.ones((_SP, 1), jnp.float32)
    for gix in range(_B):
        f = feat_ref[gix]
        fm = jnp.where(feat_on, f, 0.0)
        mu = jnp.sum(fm, axis=1, keepdims=True) * 0.2
        var = jnp.sum(jnp.where(feat_on, (fm - mu) ** 2, 0.0),
                      axis=1, keepdims=True) * 0.2
        nf = (fm - mu) * jax.lax.rsqrt(var + 1e-5) * lng_ref[...] \
            + lnb_ref[...]
        nf = jnp.where(feat_on, nf, 0.0)

        mask = g_ref[gix] > 0.0
        Tm = t_ref[gix]

        ft0 = jnp.dot(nf, w0_ref[...], preferred_element_type=jnp.float32)
        h0 = _gat(ft0, al0_ref[...], ar0_ref[...], ae0_ref[...],
                  we0_ref[...], b0_ref[...], _F0, mask, Tm, ones_col)
        ft1 = jnp.dot(h0, w1_ref[...], preferred_element_type=jnp.float32)
        h1 = _gat(ft1, al1_ref[...], ar1_ref[...], ae1_ref[...],
                  we1_ref[...], b1_ref[...], _ED, mask, Tm, ones_col)
        out_ref[gix] = h1


def _final_body(x_ref, w_ref, b_ref, out_ref):
    k = pl.program_id(0)
    acc = jnp.zeros((x_ref.shape[0], _ED), jnp.float32)
    for n in range(8):
        acc = acc + jnp.dot(x_ref[:, n, :], w_ref[n],
                            preferred_element_type=jnp.float32)

    @pl.when(k == 0)
    def _():
        out_ref[...] = acc

    @pl.when(k > 0)
    def _():
        out_ref[...] = out_ref[...] + acc

    @pl.when(k == (_N // 8) - 1)
    def _():
        out_ref[...] = _lrelu(out_ref[...] + b_ref[...], 0.01)


def kernel(Graph, norm_h, norm_L, norm_W, norm_P, norm_N, T, ln_g, ln_b,
           W0, We0, al0, ar0, ae0, b0, W1, We1, al1, ar1, ae1, b1, Wl, bl):
    bs = Graph.shape[0]
    G = Graph.reshape(bs, _J, _N)
    Gp = jnp.pad(G, ((0, 0), (0, _SP - _J), (0, _SP - _N)))
    Tp = jnp.pad(T, ((0, 0), (0, _SP - _J), (0, _SP - _J)))
    other = jnp.concatenate([norm_W, norm_P, norm_N], axis=1)
    jobf = jnp.concatenate(
        [norm_h[:, :, None], norm_L[:, :, None],
         jnp.broadcast_to(other[:, None, :], (bs, _J, 3))], axis=2)
    feats = jnp.pad(jobf, ((0, 0), (0, _SP - _J), (0, 3)))
    lng = jnp.pad(ln_g, (0, 3)).reshape(1, 8)
    lnb = jnp.pad(ln_b, (0, 3)).reshape(1, 8)
    w0 = jnp.pad(W0, ((0, 3), (0, 0)))
    we0 = We0.reshape(_H, _F0)
    b0r = b0.reshape(_H, _F0)
    we1 = We1.reshape(_H, _ED)
    b1r = b1.reshape(_H, _ED)

    h1 = pl.pallas_call(
        _gnn_body,
        grid=(bs // _B,),
        in_specs=[
            pl.BlockSpec((_B, _SP, 8), lambda i: (i, 0, 0)),
            pl.BlockSpec((_B, _SP, _SP), lambda i: (i, 0, 0)),
            pl.BlockSpec((_B, _SP, _SP), lambda i: (i, 0, 0)),
            pl.BlockSpec((1, 8), lambda i: (0, 0)),
            pl.BlockSpec((1, 8), lambda i: (0, 0)),
            pl.BlockSpec((8, _H * _F0), lambda i: (0, 0)),
            pl.BlockSpec((_H, _F0), lambda i: (0, 0)),
            pl.BlockSpec((_H, _F0), lambda i: (0, 0)),
            pl.BlockSpec((_H, _F0), lambda i: (0, 0)),
            pl.BlockSpec((_H, _F0), lambda i: (0, 0)),
            pl.BlockSpec((_H, _F0), lambda i: (0, 0)),
            pl.BlockSpec((_F0, _H * _ED), lambda i: (0, 0)),
            pl.BlockSpec((_H, _ED), lambda i: (0, 0)),
            pl.BlockSpec((_H, _ED), lambda i: (0, 0)),
            pl.BlockSpec((_H, _ED), lambda i: (0, 0)),
            pl.BlockSpec((_H, _ED), lambda i: (0, 0)),
            pl.BlockSpec((_H, _ED), lambda i: (0, 0)),
        ],
        out_specs=pl.BlockSpec((_B, _SP, _SP), lambda i: (i, 0, 0)),
        out_shape=jax.ShapeDtypeStruct((bs, _SP, _SP), jnp.float32),
    )(feats, Gp, Tp, lng, lnb, w0, al0, ar0, ae0, we0, b0r,
      W1, al1, ar1, ae1, we1, b1r)

    Wlr = Wl.reshape(_N, _ED, _ED)
    out = pl.pallas_call(
        _final_body,
        grid=(_N // 8,),
        in_specs=[
            pl.BlockSpec((bs, 8, _ED), lambda k: (0, k, 0)),
            pl.BlockSpec((8, _ED, _ED), lambda k: (k, 0, 0)),
            pl.BlockSpec((1, _ED), lambda k: (0, 0)),
        ],
        out_specs=pl.BlockSpec((bs, _ED), lambda k: (0, 0)),
        out_shape=jax.ShapeDtypeStruct((bs, _ED), jnp.float32),
    )(h1, Wlr, bl.reshape(1, _ED))
    return out
